# baseline (speedup 1.0000x reference)
import jax
import jax.numpy as jnp
from jax.experimental import pallas as pl
from jax.experimental.pallas import tpu as pltpu

_B = 64
_N = 100000
_CH = 2048
_NFULL = _N // _CH
_T1 = 1536
_T2 = 128
_NBUF = 16
_CAP = 1e-6
_LN2 = 0.6931471805599453


def _body(p_hbm, m_hbm, loss_ref, tgt_ref, non_ref,
          pbuf, mbuf, pt1, mt1, pt2, mt2, psem, msem, tsem):
    def chunk_copies(j):
        b = j % _NBUF
        cols = pl.ds(j * _CH, _CH)
        out = []
        for h in range(2):
            r = pl.ds(h * (_B // 2), _B // 2)
            out.append(pltpu.make_async_copy(
                p_hbm.at[r, cols], pbuf.at[b, r], psem.at[b]))
            out.append(pltpu.make_async_copy(
                m_hbm.at[r, cols], mbuf.at[b, r], msem.at[b]))
        return out

    def start(j):
        for cp in chunk_copies(j):
            cp.start()

    def wait(j):
        for cp in chunk_copies(j):
            cp.wait()

    t0 = _NFULL * _CH
    pltpu.make_async_copy(p_hbm.at[:, pl.ds(t0, _T1)], pt1, tsem.at[0]).start()
    pltpu.make_async_copy(m_hbm.at[:, pl.ds(t0, _T1)], mt1, tsem.at[1]).start()
    pltpu.make_async_copy(p_hbm.at[:, pl.ds(t0 + _T1, _T2)], pt2, tsem.at[2]).start()
    pltpu.make_async_copy(m_hbm.at[:, pl.ds(t0 + _T1, _T2)], mt2, tsem.at[3]).start()

    for j in range(_NBUF):
        start(j)

    def accum_slice(p, raw, acc):
        a_all, a_tgt, a_cnt = acc
        mf = raw.astype(jnp.float32)
        t = jnp.where(mf > 0.0, p, 1.0 - p)
        l = jnp.log2(jnp.maximum(t, _CAP))
        return (a_all + l, a_tgt + l * mf, a_cnt + mf)

    acc = (jnp.zeros((_B, 128), jnp.float32),
           jnp.zeros((_B, 128), jnp.float32),
           jnp.zeros((_B, 128), jnp.float32))
    for j in range(_NFULL):
        b = j % _NBUF
        wait(j)
        for s in range(_CH // 128):
            sl = pl.ds(s * 128, 128)
            acc = accum_slice(pbuf[b, :, sl], mbuf[b, :, sl], acc)
        nxt = j + _NBUF
        if nxt < _NFULL:
            start(nxt)

    pltpu.make_async_copy(p_hbm.at[:, pl.ds(t0, _T1)], pt1, tsem.at[0]).wait()
    pltpu.make_async_copy(m_hbm.at[:, pl.ds(t0, _T1)], mt1, tsem.at[1]).wait()
    for s in range(_T1 // 128):
        sl = pl.ds(s * 128, 128)
        acc = accum_slice(pt1[:, sl], mt1[:, sl], acc)
    a_all, a_tgt, a_cnt = acc

    pltpu.make_async_copy(p_hbm.at[:, pl.ds(t0 + _T1, _T2)], pt2, tsem.at[2]).wait()
    pltpu.make_async_copy(m_hbm.at[:, pl.ds(t0 + _T1, _T2)], mt2, tsem.at[3]).wait()
    p2 = pt2[...]
    mf2 = mt2[...].astype(jnp.float32)
    t2 = jnp.where(mf2 > 0.0, p2, 1.0 - p2)
    l2 = jnp.log2(jnp.maximum(t2, _CAP))

    n_tgt = jnp.sum(a_cnt, axis=1, keepdims=True) + jnp.sum(mf2, axis=1, keepdims=True)
    s_tgt = _LN2 * (jnp.sum(a_tgt, axis=1, keepdims=True)
                    + jnp.sum(l2 * mf2, axis=1, keepdims=True))
    s_all = _LN2 * (jnp.sum(a_all, axis=1, keepdims=True)
                    + jnp.sum(l2, axis=1, keepdims=True))
    s_non = s_all - s_tgt
    n_non = float(_N) - n_tgt
    lt = -(s_tgt / n_tgt)
    ln = -(s_non / n_non)
    loss_tgt = jnp.sum(lt) * (1.0 / _B)
    loss_non = jnp.sum(ln) * (1.0 / _B)
    loss = loss_tgt + loss_non
    loss_ref[...] = jnp.full((8, 128), loss, jnp.float32)
    tgt_ref[...] = jnp.full((8, 128), loss_tgt, jnp.float32)
    non_ref[...] = jnp.full((8, 128), loss_non, jnp.float32)


def kernel(prob_pred, mask_gt):
    outs = pl.pallas_call(
        _body,
        in_specs=[
            pl.BlockSpec(memory_space=pl.ANY),
            pl.BlockSpec(memory_space=pl.ANY),
        ],
        out_shape=[jax.ShapeDtypeStruct((8, 128), jnp.float32)] * 3,
        scratch_shapes=[
            pltpu.VMEM((_NBUF, _B, _CH), jnp.float32),
            pltpu.VMEM((_NBUF, _B, _CH), jnp.uint8),
            pltpu.VMEM((_B, _T1), jnp.float32),
            pltpu.VMEM((_B, _T1), jnp.uint8),
            pltpu.VMEM((_B, _T2), jnp.float32),
            pltpu.VMEM((_B, _T2), jnp.uint8),
            pltpu.SemaphoreType.DMA((_NBUF,)),
            pltpu.SemaphoreType.DMA((_NBUF,)),
            pltpu.SemaphoreType.DMA((4,)),
        ],
    )(prob_pred, mask_gt.view(jnp.uint8))
    loss, lt, ln = outs
    return (loss[0, 0], lt[0, 0], ln[0, 0])


# speedup vs baseline: 1.0098x; 1.0098x over previous
import jax
import jax.numpy as jnp
from jax.experimental import pallas as pl
from jax.experimental.pallas import tpu as pltpu

_B = 64
_N = 100000
_CH = 2560
_NFULL = _N // _CH
_T1 = 128
_T2 = 32
_NBUF = 12
_CAP = 1e-6
_LN2 = 0.6931471805599453


def _body(p_hbm, m_hbm, loss_ref, tgt_ref, non_ref,
          pbuf, mbuf, pt1, mt1, pt2, mt2, psem, msem, tsem):
    def start(j):
        b = j % _NBUF
        pltpu.make_async_copy(
            p_hbm.at[:, pl.ds(j * _CH, _CH)], pbuf.at[b], psem.at[b]).start()
        pltpu.make_async_copy(
            m_hbm.at[:, pl.ds(j * _CH, _CH)], mbuf.at[b], msem.at[b]).start()

    def wait(j):
        b = j % _NBUF
        pltpu.make_async_copy(
            p_hbm.at[:, pl.ds(j * _CH, _CH)], pbuf.at[b], psem.at[b]).wait()
        pltpu.make_async_copy(
            m_hbm.at[:, pl.ds(j * _CH, _CH)], mbuf.at[b], msem.at[b]).wait()

    t0 = _NFULL * _CH
    pltpu.make_async_copy(p_hbm.at[:, pl.ds(t0, _T1)], pt1, tsem.at[0]).start()
    pltpu.make_async_copy(m_hbm.at[:, pl.ds(t0, _T1)], mt1, tsem.at[1]).start()
    pltpu.make_async_copy(p_hbm.at[:, pl.ds(t0 + _T1, _T2)], pt2, tsem.at[2]).start()
    pltpu.make_async_copy(m_hbm.at[:, pl.ds(t0 + _T1, _T2)], mt2, tsem.at[3]).start()

    for j in range(_NBUF):
        start(j)

    def accum_slice(p, raw, acc):
        a_all, a_tgt, a_cnt = acc
        mf = raw.astype(jnp.float32)
        t = jnp.where(mf > 0.0, p, 1.0 - p)
        l = jnp.log2(jnp.maximum(t, _CAP))
        return (a_all + l, a_tgt + l * mf, a_cnt + mf)

    acc = (jnp.zeros((_B, 128), jnp.float32),
           jnp.zeros((_B, 128), jnp.float32),
           jnp.zeros((_B, 128), jnp.float32))
    for j in range(_NFULL):
        b = j % _NBUF
        wait(j)
        for s in range(_CH // 128):
            sl = pl.ds(s * 128, 128)
            acc = accum_slice(pbuf[b, :, sl], mbuf[b, :, sl], acc)
        nxt = j + _NBUF
        if nxt < _NFULL:
            start(nxt)

    pltpu.make_async_copy(p_hbm.at[:, pl.ds(t0, _T1)], pt1, tsem.at[0]).wait()
    pltpu.make_async_copy(m_hbm.at[:, pl.ds(t0, _T1)], mt1, tsem.at[1]).wait()
    for s in range(_T1 // 128):
        sl = pl.ds(s * 128, 128)
        acc = accum_slice(pt1[:, sl], mt1[:, sl], acc)
    a_all, a_tgt, a_cnt = acc

    pltpu.make_async_copy(p_hbm.at[:, pl.ds(t0 + _T1, _T2)], pt2, tsem.at[2]).wait()
    pltpu.make_async_copy(m_hbm.at[:, pl.ds(t0 + _T1, _T2)], mt2, tsem.at[3]).wait()
    p2 = pt2[...]
    mf2 = mt2[...].astype(jnp.float32)
    t2 = jnp.where(mf2 > 0.0, p2, 1.0 - p2)
    l2 = jnp.log2(jnp.maximum(t2, _CAP))

    n_tgt = jnp.sum(a_cnt, axis=1, keepdims=True) + jnp.sum(mf2, axis=1, keepdims=True)
    s_tgt = _LN2 * (jnp.sum(a_tgt, axis=1, keepdims=True)
                    + jnp.sum(l2 * mf2, axis=1, keepdims=True))
    s_all = _LN2 * (jnp.sum(a_all, axis=1, keepdims=True)
                    + jnp.sum(l2, axis=1, keepdims=True))
    s_non = s_all - s_tgt
    n_non = float(_N) - n_tgt
    lt = -(s_tgt / n_tgt)
    ln = -(s_non / n_non)
    loss_tgt = jnp.sum(lt) * (1.0 / _B)
    loss_non = jnp.sum(ln) * (1.0 / _B)
    loss = loss_tgt + loss_non
    loss_ref[...] = jnp.full((8, 128), loss, jnp.float32)
    tgt_ref[...] = jnp.full((8, 128), loss_tgt, jnp.float32)
    non_ref[...] = jnp.full((8, 128), loss_non, jnp.float32)


def kernel(prob_pred, mask_gt):
    outs = pl.pallas_call(
        _body,
        in_specs=[
            pl.BlockSpec(memory_space=pl.ANY),
            pl.BlockSpec(memory_space=pl.ANY),
        ],
        out_shape=[jax.ShapeDtypeStruct((8, 128), jnp.float32)] * 3,
        scratch_shapes=[
            pltpu.VMEM((_NBUF, _B, _CH), jnp.float32),
            pltpu.VMEM((_NBUF, _B, _CH), jnp.uint8),
            pltpu.VMEM((_B, _T1), jnp.float32),
            pltpu.VMEM((_B, _T1), jnp.uint8),
            pltpu.VMEM((_B, _T2), jnp.float32),
            pltpu.VMEM((_B, _T2), jnp.uint8),
            pltpu.SemaphoreType.DMA((_NBUF,)),
            pltpu.SemaphoreType.DMA((_NBUF,)),
            pltpu.SemaphoreType.DMA((4,)),
        ],
    )(prob_pred, mask_gt.view(jnp.uint8))
    loss, lt, ln = outs
    return (loss[0, 0], lt[0, 0], ln[0, 0])
